# Initial kernel scaffold; baseline (speedup 1.0000x reference)
import functools

import jax
import jax.numpy as jnp
from jax import lax
from jax.experimental import pallas as pl
from jax.experimental.pallas import tpu as pltpu
from jax.experimental.pallas import tpu_sc as plsc

N_USERS = 5000
N_ITEMS = 5000
N = N_USERS + N_ITEMS
EMB = 128
HALF = EMB // 2
LAYERS = 3
E = 320000

NC = 2
NS = 16
EPT = E // NS
CH = 400
NCHUNK = EPT // CH
RPT = N // NS


def _body(src_hbm, dst_hbm, w_hbm, x0_hbm, y1, y2, y3,
          acc, src_v, dst_v, w_s, gbuf, sem):
    c = lax.axis_index("c")
    s = lax.axis_index("s")
    r0 = s * RPT

    zeros16 = jnp.zeros((16,), jnp.float32)

    def zero_gbuf(i, carry):
        for cb in range(HALF // 16):
            gbuf[i, pl.ds(cb * 16, 16)] = zeros16
        return carry

    def layer(xin, yout):
        lax.fori_loop(0, CH, zero_gbuf, 0)
        pltpu.sync_copy(gbuf, acc.at[pl.ds(r0, CH)])
        pltpu.sync_copy(gbuf.at[pl.ds(0, RPT - CH)],
                        acc.at[pl.ds(r0 + CH, RPT - CH)])
        plsc.subcore_barrier()

        def chunk(k, carry):
            off = s * EPT + k * CH
            pltpu.sync_copy(src_hbm.at[pl.ds(off, CH)], src_v)
            pltpu.sync_copy(dst_hbm.at[pl.ds(off, CH)], dst_v)
            pltpu.sync_copy(w_hbm.at[pl.ds(off, CH)], w_s)
            pltpu.async_copy(xin.at[src_v], gbuf, sem).wait()

            def scale(e, inner):
                wv = jnp.full((16,), w_s[e])
                for cb in range(HALF // 16):
                    sl = pl.ds(cb * 16, 16)
                    gbuf[e, sl] = gbuf[e, sl] * wv
                return inner

            lax.fori_loop(0, CH, scale, 0)
            pltpu.sync_copy(gbuf, acc.at[dst_v], add=True)
            return carry

        lax.fori_loop(0, NCHUNK, chunk, 0)
        plsc.subcore_barrier()
        pltpu.sync_copy(acc.at[pl.ds(r0, RPT)], yout.at[c, pl.ds(r0, RPT)])
        plsc.subcore_barrier()

    layer(x0_hbm.at[c], y1)
    layer(y1.at[c], y2)
    layer(y2.at[c], y3)


@jax.jit
def _propagate(src, dst, w, x0_halves):
    out3 = [jax.ShapeDtypeStruct((NC, N, HALF), jnp.float32)] * LAYERS
    run = pl.kernel(
        _body,
        out_type=out3,
        mesh=plsc.VectorSubcoreMesh(core_axis_name="c", subcore_axis_name="s"),
        scratch_types=[
            pltpu.VMEM_SHARED((N, HALF), jnp.float32),
            pltpu.VMEM((CH,), jnp.int32),
            pltpu.VMEM((CH,), jnp.int32),
            pltpu.SMEM((CH,), jnp.float32),
            pltpu.VMEM((CH, HALF), jnp.float32),
            pltpu.SemaphoreType.DMA,
        ],
    )
    return run(src, dst, w, x0_halves)


def kernel(edge_index, edge_weight, user_emb, item_emb):
    x0 = jnp.concatenate([user_emb, item_emb], axis=0)
    x0_halves = jnp.stack([x0[:, :HALF], x0[:, HALF:]], axis=0)
    src = edge_index[0].astype(jnp.int32)
    dst = edge_index[1].astype(jnp.int32)
    w = edge_weight.astype(jnp.float32)
    ys = _propagate(src, dst, w, x0_halves)
    outs = tuple(y.transpose(1, 0, 2).reshape(N, EMB) for y in ys)
    return (x0,) + outs


# speedup vs baseline: 2.9352x; 2.9352x over previous
import functools

import jax
import jax.numpy as jnp
from jax import lax
from jax.experimental import pallas as pl
from jax.experimental.pallas import tpu as pltpu
from jax.experimental.pallas import tpu_sc as plsc

N_USERS = 5000
N_ITEMS = 5000
N = N_USERS + N_ITEMS
EMB = 128
HALF = EMB // 2
LAYERS = 3
E = 320000

NC = 2
NS = 16
EPT = E // NS
CH = 400
NCHUNK = EPT // CH
RPT = N // NS


def _body(src_hbm, dst_hbm, w_hbm, x0_hbm, y1, y2, y3,
          acc, src_v, dst_v, w_s, gbuf, sem):
    c = lax.axis_index("c")
    s = lax.axis_index("s")
    r0 = s * RPT

    zeros16 = jnp.zeros((16,), jnp.float32)

    def zero_gbuf(i, carry):
        for cb in range(HALF // 16):
            gbuf[i, pl.ds(cb * 16, 16)] = zeros16
        return carry

    def layer(xin, yout):
        lax.fori_loop(0, CH, zero_gbuf, 0)
        pltpu.sync_copy(gbuf, acc.at[pl.ds(r0, CH)])
        pltpu.sync_copy(gbuf.at[pl.ds(0, RPT - CH)],
                        acc.at[pl.ds(r0 + CH, RPT - CH)])
        plsc.subcore_barrier()

        def chunk(k, carry):
            off = s * EPT + k * CH
            pltpu.sync_copy(src_hbm.at[pl.ds(off, CH)], src_v)
            pltpu.sync_copy(dst_hbm.at[pl.ds(off, CH)], dst_v)
            pltpu.sync_copy(w_hbm.at[pl.ds(off, CH)], w_s)
            pltpu.async_copy(xin.at[src_v], gbuf, sem).wait()

            def scale(g, inner):
                wvec = w_s[pl.ds(g * 16, 16)]
                for j in range(16):
                    e = g * 16 + j
                    wj = jnp.full((16,), wvec[j])
                    for cb in range(HALF // 16):
                        sl = pl.ds(cb * 16, 16)
                        gbuf[e, sl] = gbuf[e, sl] * wj
                return inner

            lax.fori_loop(0, CH // 16, scale, 0)
            pltpu.sync_copy(gbuf, acc.at[dst_v], add=True)
            return carry

        lax.fori_loop(0, NCHUNK, chunk, 0)
        plsc.subcore_barrier()
        pltpu.sync_copy(acc.at[pl.ds(r0, RPT)], yout.at[c, pl.ds(r0, RPT)])
        plsc.subcore_barrier()

    layer(x0_hbm.at[c], y1)
    layer(y1.at[c], y2)
    layer(y2.at[c], y3)


@jax.jit
def _propagate(src, dst, w, x0_halves):
    out3 = [jax.ShapeDtypeStruct((NC, N, HALF), jnp.float32)] * LAYERS
    run = pl.kernel(
        _body,
        out_type=out3,
        mesh=plsc.VectorSubcoreMesh(core_axis_name="c", subcore_axis_name="s"),
        scratch_types=[
            pltpu.VMEM_SHARED((N, HALF), jnp.float32),
            pltpu.VMEM((CH,), jnp.int32),
            pltpu.VMEM((CH,), jnp.int32),
            pltpu.VMEM((CH,), jnp.float32),
            pltpu.VMEM((CH, HALF), jnp.float32),
            pltpu.SemaphoreType.DMA,
        ],
        compiler_params=pltpu.CompilerParams(use_tc_tiling_on_sc=False),
    )
    return run(src, dst, w, x0_halves)


def kernel(edge_index, edge_weight, user_emb, item_emb):
    x0 = jnp.concatenate([user_emb, item_emb], axis=0)
    x0_halves = jnp.stack([x0[:, :HALF], x0[:, HALF:]], axis=0)
    src = edge_index[0].astype(jnp.int32)
    dst = edge_index[1].astype(jnp.int32)
    w = edge_weight.astype(jnp.float32)
    ys = _propagate(src, dst, w, x0_halves)
    outs = tuple(y.transpose(1, 0, 2).reshape(N, EMB) for y in ys)
    return (x0,) + outs


# speedup vs baseline: 3.2201x; 1.0970x over previous
import functools

import jax
import jax.numpy as jnp
from jax import lax
from jax.experimental import pallas as pl
from jax.experimental.pallas import tpu as pltpu
from jax.experimental.pallas import tpu_sc as plsc

N_USERS = 5000
N_ITEMS = 5000
N = N_USERS + N_ITEMS
EMB = 128
HALF = EMB // 2
LAYERS = 3
E = 320000

NC = 2
NS = 16
EPT = E // NS
CH = 800
NCHUNK = EPT // CH
RPT = N // NS


def _body(src_hbm, dst_hbm, w_hbm, x0_hbm, y1, y2, y3,
          acc, src_v, dst_v, w_s, gbuf, sem):
    c = lax.axis_index("c")
    s = lax.axis_index("s")
    r0 = s * RPT

    zeros16 = jnp.zeros((16,), jnp.float32)

    def zero_gbuf(i, carry):
        for cb in range(HALF // 16):
            gbuf[i, pl.ds(cb * 16, 16)] = zeros16
        return carry

    def layer(xin, yout):
        nz = min(CH, RPT)
        lax.fori_loop(0, nz, zero_gbuf, 0)
        done = 0
        while done < RPT:
            step = min(nz, RPT - done)
            pltpu.sync_copy(gbuf.at[pl.ds(0, step)],
                            acc.at[pl.ds(r0 + done, step)])
            done += step
        plsc.subcore_barrier()

        def chunk(k, carry):
            off = s * EPT + k * CH
            pltpu.sync_copy(src_hbm.at[pl.ds(off, CH)], src_v)
            pltpu.sync_copy(dst_hbm.at[pl.ds(off, CH)], dst_v)
            pltpu.sync_copy(w_hbm.at[pl.ds(off, CH)], w_s)
            pltpu.async_copy(xin.at[src_v], gbuf, sem).wait()

            def scale(g, inner):
                wvec = w_s[pl.ds(g * 16, 16)]
                for j in range(16):
                    e = g * 16 + j
                    wj = jnp.full((16,), wvec[j])
                    for cb in range(HALF // 16):
                        sl = pl.ds(cb * 16, 16)
                        gbuf[e, sl] = gbuf[e, sl] * wj
                return inner

            lax.fori_loop(0, CH // 16, scale, 0)
            pltpu.sync_copy(gbuf, acc.at[dst_v], add=True)
            return carry

        lax.fori_loop(0, NCHUNK, chunk, 0)
        plsc.subcore_barrier()
        pltpu.sync_copy(acc.at[pl.ds(r0, RPT)], yout.at[c, pl.ds(r0, RPT)])
        plsc.subcore_barrier()

    layer(x0_hbm.at[c], y1)
    layer(y1.at[c], y2)
    layer(y2.at[c], y3)


@jax.jit
def _propagate(src, dst, w, x0_halves):
    out3 = [jax.ShapeDtypeStruct((NC, N, HALF), jnp.float32)] * LAYERS
    run = pl.kernel(
        _body,
        out_type=out3,
        mesh=plsc.VectorSubcoreMesh(core_axis_name="c", subcore_axis_name="s"),
        scratch_types=[
            pltpu.VMEM_SHARED((N, HALF), jnp.float32),
            pltpu.VMEM((CH,), jnp.int32),
            pltpu.VMEM((CH,), jnp.int32),
            pltpu.VMEM((CH,), jnp.float32),
            pltpu.VMEM((CH, HALF), jnp.float32),
            pltpu.SemaphoreType.DMA,
        ],
        compiler_params=pltpu.CompilerParams(use_tc_tiling_on_sc=False),
    )
    return run(src, dst, w, x0_halves)


def kernel(edge_index, edge_weight, user_emb, item_emb):
    x0 = jnp.concatenate([user_emb, item_emb], axis=0)
    x0_halves = jnp.stack([x0[:, :HALF], x0[:, HALF:]], axis=0)
    src = edge_index[0].astype(jnp.int32)
    dst = edge_index[1].astype(jnp.int32)
    w = edge_weight.astype(jnp.float32)
    ys = _propagate(src, dst, w, x0_halves)
    outs = tuple(y.transpose(1, 0, 2).reshape(N, EMB) for y in ys)
    return (x0,) + outs


# speedup vs baseline: 3.7337x; 1.1595x over previous
import jax
import jax.numpy as jnp
from jax import lax
from jax.experimental import pallas as pl
from jax.experimental.pallas import tpu as pltpu
from jax.experimental.pallas import tpu_sc as plsc

N_USERS = 5000
N_ITEMS = 5000
N = N_USERS + N_ITEMS
EMB = 128
HALF = EMB // 2
LAYERS = 3
E = 320000

NC = 2
NS = 16
EPT = E // NS
CH = 400
NCHUNK = EPT // CH
RPT = N // NS


def _body(ed_hbm, x0_hbm, y1, y2, y3,
          acc, eb0, eb1, gb0, gb1, ie0, ie1, ge0, ge1):
    c = lax.axis_index("c")
    s = lax.axis_index("s")
    r0 = s * RPT
    ebufs, gbufs = (eb0, eb1), (gb0, gb1)
    isems, gsems = (ie0, ie1), (ge0, ge1)

    zeros16 = jnp.zeros((16,), jnp.float32)

    def zero_gbuf(i, carry):
        for cb in range(HALF // 16):
            gb0[i, pl.ds(cb * 16, 16)] = zeros16
        return carry

    def layer(xin, yout):
        lax.fori_loop(0, CH, zero_gbuf, 0)
        done = 0
        while done < RPT:
            step = min(CH, RPT - done)
            pltpu.sync_copy(gb0.at[pl.ds(0, step)],
                            acc.at[pl.ds(r0 + done, step)])
            done += step
        plsc.subcore_barrier()

        def issue_idx(kk, b):
            pltpu.async_copy(ed_hbm.at[s * NCHUNK + kk], ebufs[b], isems[b])

        def wait_idx(b):
            pltpu.make_async_copy(ed_hbm.at[0], ebufs[b], isems[b]).wait()

        def issue_gather(b):
            pltpu.async_copy(xin.at[ebufs[b].at[0]], gbufs[b], gsems[b])

        def wait_gather(b):
            pltpu.make_async_copy(xin.at[ebufs[b].at[0]], gbufs[b],
                                  gsems[b]).wait()

        issue_idx(0, 0)
        issue_idx(1, 1)
        wait_idx(0)
        issue_gather(0)

        def pair(k, carry):
            for b in range(2):
                kk = k + b
                wait_gather(b)

                @pl.when(kk + 1 < NCHUNK)
                def _():
                    wait_idx(1 - b)
                    issue_gather(1 - b)

                gbuf = gbufs[b]
                wbits = ebufs[b]

                def scale(g, inner):
                    wvec = lax.bitcast_convert_type(
                        wbits[2, pl.ds(g * 16, 16)], jnp.float32)
                    for j in range(16):
                        e = g * 16 + j
                        wj = jnp.full((16,), wvec[j])
                        for cb in range(HALF // 16):
                            sl = pl.ds(cb * 16, 16)
                            gbuf[e, sl] = gbuf[e, sl] * wj
                    return inner

                lax.fori_loop(0, CH // 16, scale, 0)
                pltpu.sync_copy(gbuf, acc.at[ebufs[b].at[1]], add=True)

                @pl.when(kk + 2 < NCHUNK)
                def _():
                    issue_idx(kk + 2, b)
            return carry

        lax.fori_loop(0, NCHUNK // 2, lambda i, cy: pair(i * 2, cy), 0)
        plsc.subcore_barrier()
        pltpu.sync_copy(acc.at[pl.ds(r0, RPT)], yout.at[c, pl.ds(r0, RPT)])
        plsc.subcore_barrier()

    layer(x0_hbm.at[c], y1)
    layer(y1.at[c], y2)
    layer(y2.at[c], y3)


@jax.jit
def _propagate(edata, x0_halves):
    out3 = [jax.ShapeDtypeStruct((NC, N, HALF), jnp.float32)] * LAYERS
    run = pl.kernel(
        _body,
        out_type=out3,
        mesh=plsc.VectorSubcoreMesh(core_axis_name="c", subcore_axis_name="s"),
        scratch_types=[
            pltpu.VMEM_SHARED((N, HALF), jnp.float32),
            pltpu.VMEM((3, CH), jnp.int32),
            pltpu.VMEM((3, CH), jnp.int32),
            pltpu.VMEM((CH, HALF), jnp.float32),
            pltpu.VMEM((CH, HALF), jnp.float32),
            pltpu.SemaphoreType.DMA,
            pltpu.SemaphoreType.DMA,
            pltpu.SemaphoreType.DMA,
            pltpu.SemaphoreType.DMA,
        ],
        compiler_params=pltpu.CompilerParams(use_tc_tiling_on_sc=False),
    )
    return run(edata, x0_halves)


def kernel(edge_index, edge_weight, user_emb, item_emb):
    x0 = jnp.concatenate([user_emb, item_emb], axis=0)
    x0_halves = jnp.stack([x0[:, :HALF], x0[:, HALF:]], axis=0)
    src = edge_index[0].astype(jnp.int32)
    dst = edge_index[1].astype(jnp.int32)
    wbits = lax.bitcast_convert_type(edge_weight.astype(jnp.float32),
                                     jnp.int32)
    edata = (jnp.stack([src, dst, wbits])
             .reshape(3, NS, NCHUNK, CH)
             .transpose(1, 2, 0, 3)
             .reshape(NS * NCHUNK, 3, CH))
    ys = _propagate(edata, x0_halves)
    outs = tuple(y.transpose(1, 0, 2).reshape(N, EMB) for y in ys)
    return (x0,) + outs


# speedup vs baseline: 8.9494x; 2.3969x over previous
import jax
import jax.numpy as jnp
from jax import lax
from jax.experimental import pallas as pl
from jax.experimental.pallas import tpu as pltpu
from jax.experimental.pallas import tpu_sc as plsc

N_USERS = 5000
N_ITEMS = 5000
N = N_USERS + N_ITEMS
EMB = 128
HALF = EMB // 2
LAYERS = 3
E = 320000

NC = 2
NS = 16
EPT = E // NS
CH = 400
NCHUNK = EPT // CH
RPT = N // NS


def _body(ed_hbm, x0_hbm, y1, y2, y3,
          acc, eb0, eb1, gb0, gb1, ie0, ie1, ge0, ge1):
    c = lax.axis_index("c")
    s = lax.axis_index("s")
    r0 = s * RPT
    ebufs, gbufs = (eb0, eb1), (gb0, gb1)
    isems, gsems = (ie0, ie1), (ge0, ge1)

    zeros16 = jnp.zeros((16,), jnp.float32)

    def zero_gbuf(i, carry):
        for cb in range(HALF // 16):
            gb0[i, pl.ds(cb * 16, 16)] = zeros16
        return carry

    def layer(xin, yout):
        lax.fori_loop(0, CH, zero_gbuf, 0)
        done = 0
        while done < RPT:
            step = min(CH, RPT - done)
            pltpu.sync_copy(gb0.at[pl.ds(0, step)],
                            acc.at[pl.ds(r0 + done, step)])
            done += step
        plsc.subcore_barrier()

        def issue_idx(kk, b):
            pltpu.async_copy(ed_hbm.at[s * NCHUNK + kk], ebufs[b], isems[b])

        def wait_idx(b):
            pltpu.make_async_copy(ed_hbm.at[0], ebufs[b], isems[b]).wait()

        def issue_gather(b):
            pltpu.async_copy(xin.at[ebufs[b].at[0]], gbufs[b], gsems[b])

        def wait_gather(b):
            pltpu.make_async_copy(xin.at[ebufs[b].at[0]], gbufs[b],
                                  gsems[b]).wait()

        issue_idx(0, 0)
        issue_idx(1, 1)
        wait_idx(0)
        issue_gather(0)

        def pair(k, carry):
            for b in range(2):
                kk = k + b
                wait_gather(b)

                @pl.when(kk + 1 < NCHUNK)
                def _():
                    wait_idx(1 - b)
                    issue_gather(1 - b)

                gbuf = gbufs[b]
                wbits = ebufs[b]

                def scale(g, inner):
                    wvec = lax.bitcast_convert_type(
                        wbits[2, pl.ds(g * 16, 16)], jnp.float32)
                    for j in range(16):
                        e = g * 16 + j
                        wj = wvec.at[jnp.full((16,), j, jnp.int32)].get(
                            mode="promise_in_bounds")
                        for cb in range(HALF // 16):
                            sl = pl.ds(cb * 16, 16)
                            gbuf[e, sl] = gbuf[e, sl] * wj
                    return inner

                lax.fori_loop(0, CH // 16, scale, 0, unroll=2)
                pltpu.sync_copy(gbuf, acc.at[ebufs[b].at[1]], add=True)

                @pl.when(kk + 2 < NCHUNK)
                def _():
                    issue_idx(kk + 2, b)
            return carry

        lax.fori_loop(0, NCHUNK // 2, lambda i, cy: pair(i * 2, cy), 0)
        plsc.subcore_barrier()
        pltpu.sync_copy(acc.at[pl.ds(r0, RPT)], yout.at[c, pl.ds(r0, RPT)])
        plsc.subcore_barrier()

    layer(x0_hbm.at[c], y1)
    layer(y1.at[c], y2)
    layer(y2.at[c], y3)


@jax.jit
def _propagate(edata, x0_halves):
    out3 = [jax.ShapeDtypeStruct((NC, N, HALF), jnp.float32)] * LAYERS
    run = pl.kernel(
        _body,
        out_type=out3,
        mesh=plsc.VectorSubcoreMesh(core_axis_name="c", subcore_axis_name="s"),
        scratch_types=[
            pltpu.VMEM_SHARED((N, HALF), jnp.float32),
            pltpu.VMEM((3, CH), jnp.int32),
            pltpu.VMEM((3, CH), jnp.int32),
            pltpu.VMEM((CH, HALF), jnp.float32),
            pltpu.VMEM((CH, HALF), jnp.float32),
            pltpu.SemaphoreType.DMA,
            pltpu.SemaphoreType.DMA,
            pltpu.SemaphoreType.DMA,
            pltpu.SemaphoreType.DMA,
        ],
        compiler_params=pltpu.CompilerParams(use_tc_tiling_on_sc=False),
    )
    return run(edata, x0_halves)


def kernel(edge_index, edge_weight, user_emb, item_emb):
    x0 = jnp.concatenate([user_emb, item_emb], axis=0)
    x0_halves = jnp.stack([x0[:, :HALF], x0[:, HALF:]], axis=0)
    src = edge_index[0].astype(jnp.int32)
    dst = edge_index[1].astype(jnp.int32)
    wbits = lax.bitcast_convert_type(edge_weight.astype(jnp.float32),
                                     jnp.int32)
    edata = (jnp.stack([src, dst, wbits])
             .reshape(3, NS, NCHUNK, CH)
             .transpose(1, 2, 0, 3)
             .reshape(NS * NCHUNK, 3, CH))
    ys = _propagate(edata, x0_halves)
    outs = tuple(y.transpose(1, 0, 2).reshape(N, EMB) for y in ys)
    return (x0,) + outs
